# Initial kernel scaffold; baseline (speedup 1.0000x reference)
import jax
import jax.numpy as jnp
from jax.experimental import pallas as pl


def kernel(q, k, v, Wq, bq, Wk, bk, Wv, bv, Wo, bo):
    raise NotImplementedError("write your pallas kernel here")


# speedup vs baseline: 2.9009x; 2.9009x over previous
import jax
import jax.numpy as jnp
from jax.experimental import pallas as pl

HIDDEN = 1024
HEADS = 16
ATT = HIDDEN // HEADS
SCALE = ATT ** -0.5


def _mha_kernel(q_ref, k_ref, v_ref, wq_ref, bq_ref, wk_ref, wv_ref, bv_ref,
                wo_ref, bo_ref, out_ref):
    qh = jax.lax.dot_general(q_ref[0], wq_ref[...], (((1,), (0,)), ((), ())),
                             preferred_element_type=jnp.float32) + bq_ref[...]
    col_head = jax.lax.broadcasted_iota(jnp.int32, (HEADS, HIDDEN), 1) // ATT
    row_head = jax.lax.broadcasted_iota(jnp.int32, (HEADS, HIDDEN), 0)
    mask = (col_head == row_head).astype(jnp.float32)
    qh_heads = qh * mask
    u = jax.lax.dot_general(qh_heads, wk_ref[...], (((1,), (1,)), ((), ())),
                            preferred_element_type=jnp.float32)
    k = k_ref[0]
    scores = jax.lax.dot_general(k, u, (((1,), (1,)), ((), ())),
                                 preferred_element_type=jnp.float32) * SCALE
    m = jnp.max(scores, axis=0, keepdims=True)
    e = jnp.exp(scores - m)
    attn = e / jnp.sum(e, axis=0, keepdims=True)
    vv = v_ref[0]
    a = jax.lax.dot_general(attn, vv, (((0,), (0,)), ((), ())),
                            preferred_element_type=jnp.float32)
    f = jax.lax.dot_general(a, wv_ref[...], (((1,), (0,)), ((), ())),
                            preferred_element_type=jnp.float32)
    upd = jnp.sum(f * mask, axis=0, keepdims=True) + bv_ref[...]
    out_ref[0] = jax.lax.dot_general(
        upd, wo_ref[...], (((1,), (0,)), ((), ())),
        preferred_element_type=jnp.float32) + bo_ref[...]


def kernel(q, k, v, Wq, bq, Wk, bk, Wv, bv, Wo, bo):
    del bk
    B = q.shape[0]
    L_K = k.shape[1]
    grid = (B,)
    full = lambda b: (0, 0)
    in_specs = [
        pl.BlockSpec((1, 1, HIDDEN), lambda b: (b, 0, 0)),
        pl.BlockSpec((1, L_K, HIDDEN), lambda b: (b, 0, 0)),
        pl.BlockSpec((1, L_K, HIDDEN), lambda b: (b, 0, 0)),
        pl.BlockSpec((HIDDEN, HIDDEN), full),
        pl.BlockSpec((1, HIDDEN), full),
        pl.BlockSpec((HIDDEN, HIDDEN), full),
        pl.BlockSpec((HIDDEN, HIDDEN), full),
        pl.BlockSpec((1, HIDDEN), full),
        pl.BlockSpec((HIDDEN, HIDDEN), full),
        pl.BlockSpec((1, HIDDEN), full),
    ]
    out = pl.pallas_call(
        _mha_kernel,
        grid=grid,
        in_specs=in_specs,
        out_specs=pl.BlockSpec((1, 1, HIDDEN), lambda b: (b, 0, 0)),
        out_shape=jax.ShapeDtypeStruct((B, 1, HIDDEN), jnp.float32),
    )(q.reshape(B, 1, HIDDEN), k, v, Wq, bq.reshape(1, HIDDEN), Wk, Wv,
      bv.reshape(1, HIDDEN), Wo, bo.reshape(1, HIDDEN))
    return out.reshape(B, HIDDEN)


# speedup vs baseline: 3.0955x; 1.0671x over previous
import jax
import jax.numpy as jnp
from jax.experimental import pallas as pl
from jax.experimental.pallas import tpu as pltpu

HIDDEN = 1024
HEADS = 16
ATT = HIDDEN // HEADS
SCALE = ATT ** -0.5


def _mha_kernel(q_ref, k_ref, v_ref, wq_ref, bq_ref, wk_ref, wv_ref, bv_ref,
                wo_ref, bo_ref, out_ref, u_ref):
    b = pl.program_id(0)
    B = q_ref.shape[0]

    @pl.when(b == 0)
    def _prologue():
        qh = jax.lax.dot_general(q_ref[...], wq_ref[...],
                                 (((1,), (0,)), ((), ())),
                                 preferred_element_type=jnp.float32)
        qh = (qh + bq_ref[...]) * SCALE
        for h in range(HEADS):
            qs = qh[:, h * ATT:(h + 1) * ATT]
            ws = wk_ref[:, h * ATT:(h + 1) * ATT]
            u_h = jax.lax.dot_general(qs, ws, (((1,), (1,)), ((), ())),
                                      preferred_element_type=jnp.float32)
            u_ref[:, h, :] = u_h

    u = u_ref[b]
    k = k_ref[0]
    scores = jax.lax.dot_general(k, u, (((1,), (1,)), ((), ())),
                                 preferred_element_type=jnp.float32)
    m = jnp.max(scores, axis=0, keepdims=True)
    e = jnp.exp(scores - m)
    attn = e * (1.0 / jnp.sum(e, axis=0, keepdims=True))
    vv = v_ref[0]
    a = jax.lax.dot_general(attn, vv, (((0,), (0,)), ((), ())),
                            preferred_element_type=jnp.float32)
    f = jax.lax.dot_general(a, wv_ref[...], (((1,), (0,)), ((), ())),
                            preferred_element_type=jnp.float32)
    col_head = jax.lax.broadcasted_iota(jnp.int32, (HEADS, HIDDEN), 1) // ATT
    row_head = jax.lax.broadcasted_iota(jnp.int32, (HEADS, HIDDEN), 0)
    mask = (col_head == row_head).astype(jnp.float32)
    upd = jnp.sum(f * mask, axis=0, keepdims=True) + bv_ref[...]
    out_ref[0] = jax.lax.dot_general(
        upd, wo_ref[...], (((1,), (0,)), ((), ())),
        preferred_element_type=jnp.float32) + bo_ref[...]


def kernel(q, k, v, Wq, bq, Wk, bk, Wv, bv, Wo, bo):
    del bk
    B = q.shape[0]
    L_K = k.shape[1]
    full = lambda b: (0, 0)
    in_specs = [
        pl.BlockSpec((B, HIDDEN), full),
        pl.BlockSpec((1, L_K, HIDDEN), lambda b: (b, 0, 0)),
        pl.BlockSpec((1, L_K, HIDDEN), lambda b: (b, 0, 0)),
        pl.BlockSpec((HIDDEN, HIDDEN), full),
        pl.BlockSpec((1, HIDDEN), full),
        pl.BlockSpec((HIDDEN, HIDDEN), full),
        pl.BlockSpec((HIDDEN, HIDDEN), full),
        pl.BlockSpec((1, HIDDEN), full),
        pl.BlockSpec((HIDDEN, HIDDEN), full),
        pl.BlockSpec((1, HIDDEN), full),
    ]
    out = pl.pallas_call(
        _mha_kernel,
        grid=(B,),
        in_specs=in_specs,
        out_specs=pl.BlockSpec((1, 1, HIDDEN), lambda b: (b, 0, 0)),
        out_shape=jax.ShapeDtypeStruct((B, 1, HIDDEN), jnp.float32),
        scratch_shapes=[pltpu.VMEM((B, HEADS, HIDDEN), jnp.float32)],
    )(q, k, v, Wq, bq.reshape(1, HIDDEN), Wk, Wv,
      bv.reshape(1, HIDDEN), Wo, bo.reshape(1, HIDDEN))
    return out.reshape(B, HIDDEN)
